# Initial kernel scaffold; baseline (speedup 1.0000x reference)
import jax
import jax.numpy as jnp
from jax.experimental import pallas as pl


def kernel(inputs):
    raise NotImplementedError("write your pallas kernel here")


# speedup vs baseline: 79.8056x; 79.8056x over previous
import functools
import math

import jax
import jax.numpy as jnp
from jax.experimental import pallas as pl

_ACT_SPARSITY = 0.65
_INT32_MIN = jnp.iinfo(jnp.int32).min
_INT32_MAX = jnp.iinfo(jnp.int32).max


def _topk_mask_kernel(x_ref, o_ref, *, k):
    x = x_ref[...]
    u = jax.lax.bitcast_convert_type(x, jnp.int32)
    key = jnp.where(u < 0, u ^ jnp.int32(0x7FFFFFFF), u)

    rows = x.shape[0]
    lo0 = jnp.full((rows, 1), _INT32_MIN, dtype=jnp.int32)
    hi0 = jnp.full((rows, 1), _INT32_MAX, dtype=jnp.int32)

    def body(_, carry):
        lo, hi = carry
        xor = lo ^ hi
        mid = (lo & hi) + (xor >> 1) + (xor & 1)
        cnt = jnp.sum((key >= mid).astype(jnp.int32), axis=1, keepdims=True)
        ge = cnt >= k
        return jnp.where(ge, mid, lo), jnp.where(ge, hi, mid - 1)

    lo, _ = jax.lax.fori_loop(0, 32, body, (lo0, hi0))
    o_ref[...] = jnp.where(key >= lo, x, 0.0)


def kernel(inputs):
    out_shape = inputs.shape
    x = inputs.reshape(inputs.shape[0], -1)
    m, n = x.shape
    k = math.floor((1.0 - _ACT_SPARSITY) * n)

    block = 256
    while m % block:
        block //= 2
    grid = m // block

    out = pl.pallas_call(
        functools.partial(_topk_mask_kernel, k=k),
        grid=(grid,),
        in_specs=[pl.BlockSpec((block, n), lambda i: (i, 0))],
        out_specs=pl.BlockSpec((block, n), lambda i: (i, 0)),
        out_shape=jax.ShapeDtypeStruct((m, n), x.dtype),
    )(x)
    return out.reshape(out_shape)
